# baseline (speedup 1.0000x reference)
import functools

import jax
import jax.numpy as jnp
from jax import lax
from jax.experimental import pallas as pl
from jax.experimental.pallas import tpu as pltpu
from jax.experimental.pallas import tpu_sc as plsc

_B = 1024
_L = 512
_M = 64
_LANES = 16

_info = plsc.get_sparse_core_info()
_NC = _info.num_cores
_NS = _info.num_subcores
_NW = _NC * _NS
_SPT = _B // _NW


def _sc_body(x_hbm, table_hbm, out_hbm, xs_v, tab_v, out_v):
    wid = lax.axis_index("s") * _NC + lax.axis_index("c")
    base = wid * _SPT

    pltpu.sync_copy(table_hbm, tab_v)
    pltpu.sync_copy(x_hbm.at[pl.ds(base * _L, _SPT * _L)], xs_v)

    lane = lax.iota(jnp.int32, 16)
    ones = jnp.ones((_LANES,), jnp.float32)

    for g in range(_SPT // _LANES):

        def sample_body(j, outvec, g=g):
            i = g * _LANES + j

            def group_body(q, accs):
                a0, a1, a2, a3 = accs
                xv = xs_v[pl.ds(i * _L + q * _LANES, _LANES)]
                for k in range(_LANES):
                    rb = (2 * _LANES * q + 2 * k + xv[k]) * _M
                    a0 = a0 * tab_v[pl.ds(rb, 16)]
                    a1 = a1 * tab_v[pl.ds(rb + 16, 16)]
                    a2 = a2 * tab_v[pl.ds(rb + 32, 16)]
                    a3 = a3 * tab_v[pl.ds(rb + 48, 16)]
                return (a0, a1, a2, a3)

            a0, a1, a2, a3 = lax.fori_loop(
                0, _L // _LANES, group_body, (ones, ones, ones, ones))
            tot = jnp.sum((a0 + a1) + (a2 + a3), axis=0)
            return jnp.where(lane == j, tot, outvec)

        outvec = lax.fori_loop(0, _LANES, sample_body,
                               jnp.zeros((_LANES,), jnp.float32))
        out_v[pl.ds(g * _LANES, _LANES)] = outvec

    pltpu.sync_copy(out_v, out_hbm.at[pl.ds(base, _SPT)])


@jax.jit
def _qgps_sc(xflat, table):
    run = pl.kernel(
        _sc_body,
        out_type=jax.ShapeDtypeStruct((_B,), jnp.float32),
        mesh=plsc.VectorSubcoreMesh(core_axis_name="c", subcore_axis_name="s"),
        scratch_types=[
            pltpu.VMEM((_SPT * _L,), jnp.int32),
            pltpu.VMEM((2 * _L * _M,), jnp.float32),
            pltpu.VMEM((_SPT,), jnp.float32),
        ],
        compiler_params=pltpu.CompilerParams(needs_layout_passes=False),
    )
    return run(xflat, table)


def kernel(inputs, epsilon):
    xflat = inputs.astype(jnp.int32).reshape(-1)
    table = jnp.transpose(epsilon, (2, 0, 1)).reshape(-1)
    return _qgps_sc(xflat, table)
